# baseline (speedup 1.0000x reference)
import functools

import jax
import jax.numpy as jnp
from jax import lax
from jax.experimental import pallas as pl
from jax.experimental.pallas import tpu as pltpu
from jax.experimental.pallas import tpu_sc as plsc

_SCALE = 8.0

_info = plsc.get_sparse_core_info()
_NC, _NS, _L = _info.num_cores, _info.num_subcores, _info.num_lanes
_NW = _NC * _NS


@functools.lru_cache(maxsize=None)
def _make_lookup(S, B, V, D):
    DG = 4
    SG = _NW // DG
    d_per_w = D // DG
    s_per_w = S // SG
    assert D % DG == 0 and S % SG == 0
    mesh = plsc.VectorSubcoreMesh(core_axis_name="c", subcore_axis_name="s")

    @functools.partial(
        pl.kernel,
        mesh=mesh,
        compiler_params=pltpu.CompilerParams(
            needs_layout_passes=False, use_tc_tiling_on_sc=False
        ),
        out_type=jax.ShapeDtypeStruct((S, D, B), jnp.float32),
        scratch_types=[
            pltpu.VMEM((B,), jnp.int32),
            pltpu.VMEM((d_per_w, B), jnp.float32),
            pltpu.SemaphoreType.DMA,
        ],
    )
    def k(xT_hbm, tT_hbm, out_hbm, idx_v, bufs_v, sem):
        wid = lax.axis_index("s") * _NC + lax.axis_index("c")
        a = wid // DG
        db = (wid % DG) * d_per_w

        def s1_body(j, carry):
            s1 = a * s_per_w + j
            pltpu.sync_copy(xT_hbm.at[s1], idx_v)
            for dd in range(d_per_w):
                pltpu.async_copy(
                    tT_hbm.at[db + dd].at[idx_v], bufs_v.at[dd], sem
                )
            pltpu.make_async_copy(
                tT_hbm.at[pl.ds(0, d_per_w), pl.ds(0, B)], bufs_v, sem
            ).wait()

            def scale_body(r, carry2):
                for j2 in range(B // _L):
                    sl = pl.ds(j2 * _L, _L)
                    bufs_v[r, sl] = bufs_v[r, sl] * _SCALE
                return carry2

            lax.fori_loop(0, d_per_w, scale_body, 0)
            pltpu.sync_copy(bufs_v, out_hbm.at[s1, pl.ds(db, d_per_w)])
            return carry

        lax.fori_loop(0, s_per_w, s1_body, 0)

    return k


def kernel(x, table):
    s0, s1 = x.shape
    V, D = table.shape
    xT = x.T.astype(jnp.int32)
    tT = table.T
    out = _make_lookup(s1, s0, V, D)(xT, tT)
    return out.transpose(2, 0, 1)


# speedup vs baseline: 10.9763x; 10.9763x over previous
import functools

import jax
import jax.numpy as jnp
from jax import lax
from jax.experimental import pallas as pl
from jax.experimental.pallas import tpu as pltpu
from jax.experimental.pallas import tpu_sc as plsc

_SCALE = 8.0

_info = plsc.get_sparse_core_info()
_NC, _NS, _L = _info.num_cores, _info.num_subcores, _info.num_lanes
_NW = _NC * _NS


@functools.lru_cache(maxsize=None)
def _make_lookup(B, V, D, chunk):
    b_per_w = B // _NW
    n_chunks = b_per_w // chunk
    assert b_per_w % chunk == 0 and chunk % _L == 0 and D % _L == 0
    mesh = plsc.VectorSubcoreMesh(core_axis_name="c", subcore_axis_name="s")

    @functools.partial(
        pl.kernel,
        mesh=mesh,
        compiler_params=pltpu.CompilerParams(needs_layout_passes=False),
        out_type=jax.ShapeDtypeStruct((B, D), jnp.float32),
        scratch_types=[
            pltpu.VMEM((chunk,), jnp.int32),
            pltpu.VMEM((chunk, D), jnp.float32),
            pltpu.SemaphoreType.DMA,
        ],
    )
    def k(idx_hbm, table_hbm, out_hbm, idx_v, rows_v, sem):
        out2 = out_hbm
        wid = lax.axis_index("s") * _NC + lax.axis_index("c")
        base = wid * b_per_w
        lane = lax.iota(jnp.int32, _L)

        def chunk_body(c, carry):
            off = base + c * chunk
            pltpu.sync_copy(idx_hbm.at[pl.ds(off, chunk)], idx_v)

            def fire_block(kk, carry2):
                v = idx_v[pl.ds(kk * _L, _L)]
                for j in range(_L):
                    row = jnp.sum(jnp.where(lane == j, v, 0))
                    pltpu.async_copy(
                        table_hbm.at[pl.ds(row, 1)],
                        rows_v.at[pl.ds(kk * _L + j, 1)],
                        sem,
                    )
                return carry2

            lax.fori_loop(0, chunk // _L, fire_block, 0)
            pltpu.make_async_copy(
                table_hbm.at[pl.ds(0, chunk)], rows_v, sem
            ).wait()

            def row_body(r, carry2):
                for j in range(D // _L):
                    sl = pl.ds(j * _L, _L)
                    rows_v[r, sl] = rows_v[r, sl] * _SCALE
                return carry2

            lax.fori_loop(0, chunk, row_body, 0)
            pltpu.sync_copy(rows_v, out2.at[pl.ds(off, chunk)])
            return carry

        lax.fori_loop(0, n_chunks, chunk_body, 0)

    return k


def kernel(x, table):
    s0, s1 = x.shape
    B = s0 * s1
    V, D = table.shape
    idx = x.reshape(B).astype(jnp.int32)
    out = _make_lookup(B, V, D, 640)(idx, table)
    return out.reshape(s0, s1, D)


# speedup vs baseline: 11.1924x; 1.0197x over previous
import functools

import jax
import jax.numpy as jnp
from jax import lax
from jax.experimental import pallas as pl
from jax.experimental.pallas import tpu as pltpu
from jax.experimental.pallas import tpu_sc as plsc

_SCALE = 8.0

_info = plsc.get_sparse_core_info()
_NC, _NS, _L = _info.num_cores, _info.num_subcores, _info.num_lanes
_NW = _NC * _NS


@functools.lru_cache(maxsize=None)
def _make_lookup(B, V, D, chunk):
    b_per_w = B // _NW
    n_chunks = b_per_w // chunk
    assert b_per_w % chunk == 0 and chunk % _L == 0 and D % _L == 0
    mesh = plsc.VectorSubcoreMesh(core_axis_name="c", subcore_axis_name="s")

    @functools.partial(
        pl.kernel,
        mesh=mesh,
        compiler_params=pltpu.CompilerParams(needs_layout_passes=False),
        out_type=jax.ShapeDtypeStruct((B, D), jnp.float32),
        scratch_types=[
            pltpu.VMEM((chunk,), jnp.int32),
            pltpu.VMEM((chunk,), jnp.int32),
            pltpu.VMEM((chunk, D), jnp.float32),
            pltpu.VMEM((chunk, D), jnp.float32),
            pltpu.SemaphoreType.DMA,
            pltpu.SemaphoreType.DMA,
        ],
    )
    def k(idx_hbm, table_hbm, out_hbm, idx_va, idx_vb, rows_va, rows_vb, sem_a, sem_b):
        assert n_chunks % 2 == 0
        wid = lax.axis_index("s") * _NC + lax.axis_index("c")
        base = wid * b_per_w
        lane = lax.iota(jnp.int32, _L)

        def fire(c, idx_v, rows_v, sem):
            off = base + c * chunk
            pltpu.sync_copy(idx_hbm.at[pl.ds(off, chunk)], idx_v)

            def fire_block(kk, carry2):
                v = idx_v[pl.ds(kk * _L, _L)]
                for j in range(_L):
                    row = jnp.sum(jnp.where(lane == j, v, 0))
                    pltpu.async_copy(
                        table_hbm.at[pl.ds(row, 1)],
                        rows_v.at[pl.ds(kk * _L + j, 1)],
                        sem,
                    )
                return carry2

            lax.fori_loop(0, chunk // _L, fire_block, 0)

        def finish(c, rows_v, sem):
            pltpu.make_async_copy(
                table_hbm.at[pl.ds(0, chunk)], rows_v, sem
            ).wait()

            def row_body(r, carry2):
                for j in range(D // _L):
                    sl = pl.ds(j * _L, _L)
                    rows_v[r, sl] = rows_v[r, sl] * _SCALE
                return carry2

            lax.fori_loop(0, chunk, row_body, 0)
            pltpu.sync_copy(rows_v, out_hbm.at[pl.ds(base + c * chunk, chunk)])

        fire(0, idx_va, rows_va, sem_a)

        def pair_body(p, carry):
            c0 = 2 * p
            fire(c0 + 1, idx_vb, rows_vb, sem_b)
            finish(c0, rows_va, sem_a)

            @pl.when(c0 + 2 < n_chunks)
            def _():
                fire(c0 + 2, idx_va, rows_va, sem_a)

            finish(c0 + 1, rows_vb, sem_b)
            return carry

        lax.fori_loop(0, n_chunks // 2, pair_body, 0)

    return k


def kernel(x, table):
    s0, s1 = x.shape
    B = s0 * s1
    V, D = table.shape
    idx = x.reshape(B).astype(jnp.int32)
    out = _make_lookup(B, V, D, 320)(idx, table)
    return out.reshape(s0, s1, D)


# speedup vs baseline: 11.5302x; 1.0302x over previous
import functools

import jax
import jax.numpy as jnp
from jax import lax
from jax.experimental import pallas as pl
from jax.experimental.pallas import tpu as pltpu
from jax.experimental.pallas import tpu_sc as plsc

_SCALE = 8.0

_info = plsc.get_sparse_core_info()
_NC, _NS, _L = _info.num_cores, _info.num_subcores, _info.num_lanes
_NW = _NC * _NS


@functools.lru_cache(maxsize=None)
def _make_lookup(B, V, D, chunk):
    b_per_w = B // _NW
    n_chunks = b_per_w // chunk
    assert b_per_w % chunk == 0 and chunk % _L == 0 and D % _L == 0
    mesh = plsc.VectorSubcoreMesh(core_axis_name="c", subcore_axis_name="s")

    @functools.partial(
        pl.kernel,
        mesh=mesh,
        compiler_params=pltpu.CompilerParams(needs_layout_passes=False),
        out_type=jax.ShapeDtypeStruct((B, D), jnp.float32),
        scratch_types=[
            pltpu.VMEM((chunk,), jnp.int32),
            pltpu.VMEM((chunk,), jnp.int32),
            pltpu.VMEM((chunk, D), jnp.float32),
            pltpu.VMEM((chunk, D), jnp.float32),
            pltpu.SemaphoreType.DMA,
            pltpu.SemaphoreType.DMA,
        ],
    )
    def k(idx_hbm, table_hbm, out_hbm, idx_va, idx_vb, rows_va, rows_vb, sem_a, sem_b):
        assert n_chunks % 2 == 0
        wid = lax.axis_index("s") * _NC + lax.axis_index("c")
        base = wid * b_per_w
        lane = lax.iota(jnp.int32, _L)

        def fire(c, idx_v, rows_v, sem):
            off = base + c * chunk
            pltpu.sync_copy(idx_hbm.at[pl.ds(off, chunk)], idx_v)

            def fire_block(kk, carry2):
                v = idx_v[pl.ds(kk * _L, _L)]
                for j in range(_L):
                    row = jnp.sum(jnp.where(lane == j, v, 0))
                    pltpu.async_copy(
                        table_hbm.at[pl.ds(row, 1)],
                        rows_v.at[pl.ds(kk * _L + j, 1)],
                        sem,
                    )
                return carry2

            lax.fori_loop(0, chunk // _L, fire_block, 0)

        def finish(c, rows_v, sem):
            pltpu.make_async_copy(
                table_hbm.at[pl.ds(0, chunk)], rows_v, sem
            ).wait()

            @plsc.parallel_loop(0, chunk, 1, unroll=4)
            def _(r):
                for j in range(D // _L):
                    sl = pl.ds(j * _L, _L)
                    rows_v[r, sl] = rows_v[r, sl] * _SCALE

            pltpu.sync_copy(rows_v, out_hbm.at[pl.ds(base + c * chunk, chunk)])

        fire(0, idx_va, rows_va, sem_a)

        def pair_body(p, carry):
            c0 = 2 * p
            fire(c0 + 1, idx_vb, rows_vb, sem_b)
            finish(c0, rows_va, sem_a)

            @pl.when(c0 + 2 < n_chunks)
            def _():
                fire(c0 + 2, idx_va, rows_va, sem_a)

            finish(c0 + 1, rows_vb, sem_b)
            return carry

        lax.fori_loop(0, n_chunks // 2, pair_body, 0)

    return k


def kernel(x, table):
    s0, s1 = x.shape
    B = s0 * s1
    V, D = table.shape
    idx = x.reshape(B).astype(jnp.int32)
    out = _make_lookup(B, V, D, 320)(idx, table)
    return out.reshape(s0, s1, D)
